# baseline (speedup 1.0000x reference)
import functools

import jax
import jax.numpy as jnp
from jax import lax
from jax.experimental import pallas as pl
from jax.experimental.pallas import tpu as pltpu
from jax.experimental.pallas import tpu_sc as plsc

L = 16


def _scores_body(x_ref, wa_ref, ba_ref, o_ref):
    x = x_ref[0]
    s = jnp.dot(x, wa_ref[...], preferred_element_type=jnp.float32)[:, 0] + ba_ref[0]
    o_ref[0, 0] = s


def _pool_body(w_ref, x_ref, wo_ref, bo_ref, o_ref):
    x = x_ref[0]
    w = w_ref[0]
    pooled = jnp.dot(w, x, preferred_element_type=jnp.float32)
    o_ref[0] = (
        jnp.dot(pooled, wo_ref[...], preferred_element_type=jnp.float32)
        + bo_ref[...][None, :]
    )


def _make_sc_softmax(B, npg):
    nv = npg // L
    mesh = plsc.VectorSubcoreMesh(
        core_axis_name="c", subcore_axis_name="s", num_cores=2, num_subcores=16
    )

    @functools.partial(
        pl.kernel,
        out_type=jax.ShapeDtypeStruct((B, npg), jnp.float32),
        mesh=mesh,
        scratch_types=[
            pltpu.VMEM((npg,), jnp.float32),
            pltpu.VMEM((npg,), jnp.float32),
            pltpu.SemaphoreType.DMA,
        ],
    )
    def sc_softmax(scores_hbm, w_hbm, s_v, e_v, sem):
        wid = lax.axis_index("s") * 2 + lax.axis_index("c")

        def lane_reduce(vec, op):
            acc = vec[0]
            for i in range(1, L):
                acc = op(acc, vec[i])
            return acc

        @pl.when(wid < B)
        def _():
            pltpu.async_copy(scores_hbm.at[wid], s_v, sem).wait()

            def max_body(i, acc):
                return jnp.maximum(acc, s_v[pl.ds(i * L, L)])

            macc = lax.fori_loop(
                0, nv, max_body, jnp.full((L,), -jnp.inf, jnp.float32)
            )
            mv = jnp.full((L,), lane_reduce(macc, jnp.maximum), jnp.float32)

            def exp_body(i, acc):
                e = jnp.exp(s_v[pl.ds(i * L, L)] - mv)
                e_v[pl.ds(i * L, L)] = e
                return acc + e

            dacc = lax.fori_loop(0, nv, exp_body, jnp.zeros((L,), jnp.float32))
            dv = jnp.full((L,), lane_reduce(dacc, jnp.add), jnp.float32)
            rv = jnp.float32(1.0) / dv

            def scale_body(i, _):
                e_v[pl.ds(i * L, L)] = e_v[pl.ds(i * L, L)] * rv
                return 0

            lax.fori_loop(0, nv, scale_body, 0)
            pltpu.async_copy(e_v, w_hbm.at[wid], sem).wait()

    return sc_softmax


def kernel(node_embeddings, batch_num_nodes, Wa, ba, Wo, bo):
    B = batch_num_nodes.shape[0]
    N, D = node_embeddings.shape
    H = Wo.shape[1]
    npg = N // B
    x3 = node_embeddings.reshape(B, npg, D)

    scores = pl.pallas_call(
        _scores_body,
        grid=(B,),
        in_specs=[
            pl.BlockSpec((1, npg, D), lambda i: (i, 0, 0)),
            pl.BlockSpec((D, 1), lambda i: (0, 0)),
            pl.BlockSpec(memory_space=pltpu.SMEM),
        ],
        out_specs=pl.BlockSpec((1, 1, npg), lambda i: (i, 0, 0)),
        out_shape=jax.ShapeDtypeStruct((B, 1, npg), jnp.float32),
    )(x3, Wa, ba).reshape(B, npg)

    w = _make_sc_softmax(B, npg)(scores)

    out = pl.pallas_call(
        _pool_body,
        grid=(B,),
        in_specs=[
            pl.BlockSpec((1, 1, npg), lambda i: (i, 0, 0)),
            pl.BlockSpec((1, npg, D), lambda i: (i, 0, 0)),
            pl.BlockSpec((D, H), lambda i: (0, 0)),
            pl.BlockSpec((H,), lambda i: (0,)),
        ],
        out_specs=pl.BlockSpec((1, 1, H), lambda i: (i, 0, 0)),
        out_shape=jax.ShapeDtypeStruct((B, 1, H), jnp.float32),
    )(w.reshape(B, 1, npg), x3, Wo, bo)
    return out.reshape(B, H)


# speedup vs baseline: 1.0324x; 1.0324x over previous
import functools

import jax
import jax.numpy as jnp
from jax import lax
from jax.experimental import pallas as pl
from jax.experimental.pallas import tpu as pltpu
from jax.experimental.pallas import tpu_sc as plsc

L = 16


def _scores_body(x_ref, wa_ref, ba_ref, o_ref):
    x = x_ref[0]
    s = jnp.sum(x * wa_ref[...][:, 0][None, :], axis=1) + ba_ref[0]
    o_ref[0, 0] = s


def _pool_body(w_ref, x_ref, wo_ref, bo_ref, o_ref, acc_ref):
    g = pl.program_id(0)
    nb = pl.num_programs(0)
    x = x_ref[0]
    w = w_ref[0, 0]
    pooled = jnp.sum(x * w[:, None], axis=0)
    acc_ref[pl.ds(g, 1), :] = pooled[None, :]

    @pl.when(g == nb - 1)
    def _():
        o_ref[...] = (
            jnp.dot(acc_ref[...], wo_ref[...], preferred_element_type=jnp.float32)
            + bo_ref[...][None, :]
        )


def _make_sc_softmax(B, npg):
    nv = npg // L
    mesh = plsc.VectorSubcoreMesh(
        core_axis_name="c", subcore_axis_name="s", num_cores=2, num_subcores=16
    )

    @functools.partial(
        pl.kernel,
        out_type=jax.ShapeDtypeStruct((B, npg), jnp.float32),
        mesh=mesh,
        scratch_types=[
            pltpu.VMEM((npg,), jnp.float32),
            pltpu.VMEM((npg,), jnp.float32),
            pltpu.SemaphoreType.DMA,
        ],
    )
    def sc_softmax(scores_hbm, w_hbm, s_v, e_v, sem):
        wid = lax.axis_index("s") * 2 + lax.axis_index("c")

        def lane_reduce(vec, op):
            acc = vec[0]
            for i in range(1, L):
                acc = op(acc, vec[i])
            return acc

        @pl.when(wid < B)
        def _():
            pltpu.async_copy(scores_hbm.at[wid], s_v, sem).wait()

            def max_body(i, acc):
                return jnp.maximum(acc, s_v[pl.ds(i * L, L)])

            macc = lax.fori_loop(
                0, nv, max_body, jnp.full((L,), -jnp.inf, jnp.float32)
            )
            mv = jnp.full((L,), lane_reduce(macc, jnp.maximum), jnp.float32)

            def exp_body(i, acc):
                e = jnp.exp(s_v[pl.ds(i * L, L)] - mv)
                e_v[pl.ds(i * L, L)] = e
                return acc + e

            dacc = lax.fori_loop(0, nv, exp_body, jnp.zeros((L,), jnp.float32))
            dv = jnp.full((L,), lane_reduce(dacc, jnp.add), jnp.float32)
            rv = jnp.float32(1.0) / dv

            def scale_body(i, _):
                e_v[pl.ds(i * L, L)] = e_v[pl.ds(i * L, L)] * rv
                return 0

            lax.fori_loop(0, nv, scale_body, 0)
            pltpu.async_copy(e_v, w_hbm.at[wid], sem).wait()

    return sc_softmax


def kernel(node_embeddings, batch_num_nodes, Wa, ba, Wo, bo):
    B = batch_num_nodes.shape[0]
    N, D = node_embeddings.shape
    H = Wo.shape[1]
    npg = N // B
    x3 = node_embeddings.reshape(B, npg, D)

    scores = pl.pallas_call(
        _scores_body,
        grid=(B,),
        in_specs=[
            pl.BlockSpec((1, npg, D), lambda i: (i, 0, 0)),
            pl.BlockSpec((D, 1), lambda i: (0, 0)),
            pl.BlockSpec(memory_space=pltpu.SMEM),
        ],
        out_specs=pl.BlockSpec((1, 1, npg), lambda i: (i, 0, 0)),
        out_shape=jax.ShapeDtypeStruct((B, 1, npg), jnp.float32),
    )(x3, Wa, ba).reshape(B, npg)

    w = _make_sc_softmax(B, npg)(scores)

    out = pl.pallas_call(
        _pool_body,
        grid=(B,),
        in_specs=[
            pl.BlockSpec((1, 1, npg), lambda i: (i, 0, 0)),
            pl.BlockSpec((1, npg, D), lambda i: (i, 0, 0)),
            pl.BlockSpec((D, H), lambda i: (0, 0)),
            pl.BlockSpec((H,), lambda i: (0,)),
        ],
        out_specs=pl.BlockSpec((B, H), lambda i: (0, 0)),
        out_shape=jax.ShapeDtypeStruct((B, H), jnp.float32),
        scratch_shapes=[pltpu.VMEM((B, H), jnp.float32)],
    )(w.reshape(B, 1, npg), x3, Wo, bo)
    return out
